# Initial kernel scaffold; baseline (speedup 1.0000x reference)
import jax
import jax.numpy as jnp
from jax.experimental import pallas as pl


def kernel(x, edge_index, edge_weight, W1, b1, W2, b2):
    raise NotImplementedError("write your pallas kernel here")


# speedup vs baseline: 8.2679x; 8.2679x over previous
import functools

import jax
import jax.numpy as jnp
from jax import lax
from jax.experimental import pallas as pl
from jax.experimental.pallas import tpu as pltpu
from jax.experimental.pallas import tpu_sc as plsc

NC = 2
NS = 16
L = 16
NTILES = NC * NS
BLK = 128
LN_EPS = 1e-5

_MESH = plsc.VectorSubcoreMesh(core_axis_name="c", subcore_axis_name="s")
_SC_PARAMS = pltpu.CompilerParams(needs_layout_passes=False)


def _zero_fill(buf, nrows, ncols):
    z = jnp.zeros((L,), jnp.float32)

    @pl.loop(0, nrows)
    def _(j):
        for k in range(ncols // L):
            buf[j, pl.ds(k * L, L)] = z


def _copy_rows(src_buf, dst, r0, total):
    full, rem = divmod(total, BLK)
    for i in range(full):
        pltpu.sync_copy(src_buf, dst.at[pl.ds(r0 + i * BLK, BLK)])
    if rem:
        pltpu.sync_copy(src_buf.at[pl.ds(0, rem)], dst.at[pl.ds(r0 + full * BLK, rem)])


def _sub_rows(n, s):
    base = (n // NS) & ~7
    last = n - base * (NS - 1)
    return s * base, base, last


def _deg_partials(colf, wf, n, e_pad):
    ept = e_pad // NTILES
    nblk = ept // BLK

    @functools.partial(
        pl.kernel,
        out_type=jax.ShapeDtypeStruct((NC, n, L), jnp.float32),
        mesh=_MESH,
        compiler_params=_SC_PARAMS,
        scratch_types=[
            pltpu.VMEM((BLK, L), jnp.float32),
            pltpu.VMEM((BLK,), jnp.int32),
            pltpu.VMEM((BLK,), jnp.float32),
            pltpu.VMEM_SHARED((n, L), jnp.float32),
        ],
    )
    def k(colf_hbm, wf_hbm, out_hbm, sbuf, icol, wbuf, acc):
        c = lax.axis_index("c")
        s = lax.axis_index("s")
        tid = c * NS + s
        r0, base, last = _sub_rows(n, s)
        _zero_fill(sbuf, BLK, L)
        _copy_rows(sbuf, acc, r0, base)

        @pl.when(s == NS - 1)
        def _():
            _copy_rows(sbuf, acc, base * NS, last - base)

        plsc.subcore_barrier()
        iota16 = lax.iota(jnp.int32, L)
        zeros16 = jnp.zeros((L,), jnp.int32)

        @pl.loop(0, nblk)
        def _(blk):
            e0 = tid * ept + blk * BLK
            pltpu.sync_copy(colf_hbm.at[pl.ds(e0, BLK)], icol)
            pltpu.sync_copy(wf_hbm.at[pl.ds(e0, BLK)], wbuf)
            for j8 in range(BLK // L):
                w16 = wbuf[pl.ds(j8 * L, L)]
                plsc.store_scatter(sbuf, [j8 * L + iota16, zeros16], w16)
            pltpu.sync_copy(sbuf, acc.at[icol], add=True)

        plsc.subcore_barrier()
        pltpu.sync_copy(acc.at[pl.ds(r0, base)], out_hbm.at[c, pl.ds(r0, base)])

        @pl.when(s == NS - 1)
        def _():
            ex = base * NS
            pltpu.sync_copy(acc.at[pl.ds(ex, last - base)],
                            out_hbm.at[c, pl.ds(ex, last - base)])

    return k(colf, wf)


def _dis_kernel(degp):
    n = degp.shape[1]

    def body(p_ref, o_ref):
        deg = jnp.sum(p_ref[...], axis=(0, 2))
        o_ref[...] = lax.rsqrt(deg)

    return pl.pallas_call(
        body,
        out_shape=jax.ShapeDtypeStruct((n,), jnp.float32),
    )(degp)


def _norm_kernel(dis, rowf, colf, wf, e_pad):
    n = dis.shape[0]
    ept = e_pad // NTILES
    nblk = ept // BLK

    @functools.partial(
        pl.kernel,
        out_type=jax.ShapeDtypeStruct((e_pad,), jnp.float32),
        mesh=_MESH,
        compiler_params=_SC_PARAMS,
        scratch_types=[
            pltpu.VMEM((n,), jnp.float32),
            pltpu.VMEM((BLK,), jnp.int32),
            pltpu.VMEM((BLK,), jnp.int32),
            pltpu.VMEM((BLK,), jnp.float32),
            pltpu.VMEM((BLK,), jnp.float32),
        ],
    )
    def k(dis_hbm, rowf_hbm, colf_hbm, wf_hbm, out_hbm, disv, irow, icol, wbuf, nbuf):
        c = lax.axis_index("c")
        s = lax.axis_index("s")
        tid = c * NS + s
        pltpu.sync_copy(dis_hbm, disv)

        @pl.loop(0, nblk)
        def _(blk):
            e0 = tid * ept + blk * BLK
            pltpu.sync_copy(rowf_hbm.at[pl.ds(e0, BLK)], irow)
            pltpu.sync_copy(colf_hbm.at[pl.ds(e0, BLK)], icol)
            pltpu.sync_copy(wf_hbm.at[pl.ds(e0, BLK)], wbuf)
            for j8 in range(BLK // L):
                sl = pl.ds(j8 * L, L)
                a = plsc.load_gather(disv, [irow[sl]])
                b = plsc.load_gather(disv, [icol[sl]])
                nbuf[sl] = a * wbuf[sl] * b
            pltpu.sync_copy(nbuf, out_hbm.at[pl.ds(e0, BLK)])

    return k(dis, rowf, colf, wf)


def _prop_kernel(h, rowf, colf, normf, e_pad):
    n, d = h.shape
    ept = e_pad // NTILES
    nblk = ept // BLK

    @functools.partial(
        pl.kernel,
        out_type=jax.ShapeDtypeStruct((NC, n, d), jnp.float32),
        mesh=_MESH,
        compiler_params=_SC_PARAMS,
        scratch_types=[
            pltpu.VMEM((BLK, d), jnp.float32),
            pltpu.VMEM((BLK,), jnp.int32),
            pltpu.VMEM((BLK,), jnp.int32),
            pltpu.VMEM((BLK,), jnp.float32),
            pltpu.VMEM_SHARED((n, d), jnp.float32),
        ],
    )
    def k(h_hbm, rowf_hbm, colf_hbm, nf_hbm, out_hbm, gbuf, irow, icol, nbuf, acc):
        c = lax.axis_index("c")
        s = lax.axis_index("s")
        tid = c * NS + s
        r0, base, last = _sub_rows(n, s)
        _zero_fill(gbuf, BLK, d)
        _copy_rows(gbuf, acc, r0, base)

        @pl.when(s == NS - 1)
        def _():
            _copy_rows(gbuf, acc, base * NS, last - base)

        plsc.subcore_barrier()

        @pl.loop(0, nblk)
        def _(blk):
            e0 = tid * ept + blk * BLK
            pltpu.sync_copy(rowf_hbm.at[pl.ds(e0, BLK)], irow)
            pltpu.sync_copy(colf_hbm.at[pl.ds(e0, BLK)], icol)
            pltpu.sync_copy(nf_hbm.at[pl.ds(e0, BLK)], nbuf)
            pltpu.sync_copy(h_hbm.at[irow], gbuf)

            @pl.loop(0, BLK)
            def _(j):
                nsplat = plsc.load_gather(nbuf, [jnp.full((L,), j, jnp.int32)])
                for k8 in range(d // L):
                    sl = pl.ds(k8 * L, L)
                    gbuf[j, sl] = gbuf[j, sl] * nsplat

            pltpu.sync_copy(gbuf, acc.at[icol], add=True)

        plsc.subcore_barrier()
        pltpu.sync_copy(acc.at[pl.ds(r0, base)], out_hbm.at[c, pl.ds(r0, base)])

        @pl.when(s == NS - 1)
        def _():
            ex = base * NS
            pltpu.sync_copy(acc.at[pl.ds(ex, last - base)],
                            out_hbm.at[c, pl.ds(ex, last - base)])

    return k(h, rowf, colf, normf)


def _matmul(x, w):
    n, d = x.shape
    blk = 1000

    def body(x_ref, w_ref, o_ref):
        o_ref[...] = jnp.dot(x_ref[...], w_ref[...],
                             preferred_element_type=jnp.float32)

    return pl.pallas_call(
        body,
        out_shape=jax.ShapeDtypeStruct((n, d), jnp.float32),
        grid=(n // blk,),
        in_specs=[
            pl.BlockSpec((blk, d), lambda i: (i, 0)),
            pl.BlockSpec((d, d), lambda i: (0, 0)),
        ],
        out_specs=pl.BlockSpec((blk, d), lambda i: (i, 0)),
    )(x, w)


def _ln_kernel(p, b):
    _, n, d = p.shape
    blk = 1000

    def body(p_ref, b_ref, o_ref):
        t = p_ref[0] + p_ref[1] + b_ref[...]
        mu = jnp.mean(t, axis=-1, keepdims=True)
        var = jnp.mean((t - mu) ** 2, axis=-1, keepdims=True)
        y = (t - mu) * lax.rsqrt(var + LN_EPS)
        o_ref[...] = jnp.maximum(y, 0.0)

    return pl.pallas_call(
        body,
        out_shape=jax.ShapeDtypeStruct((n, d), jnp.float32),
        grid=(n // blk,),
        in_specs=[
            pl.BlockSpec((2, blk, d), lambda i: (0, i, 0)),
            pl.BlockSpec((1, d), lambda i: (0, 0)),
        ],
        out_specs=pl.BlockSpec((blk, d), lambda i: (i, 0)),
    )(p, b)


def kernel(x, edge_index, edge_weight, W1, b1, W2, b2):
    n, d = x.shape
    e = edge_weight.shape[0]
    row = edge_index[0].astype(jnp.int32)
    col = edge_index[1].astype(jnp.int32)
    loop_idx = jnp.arange(n, dtype=jnp.int32)
    e_full = e + n
    chunk = NTILES * BLK
    e_pad = ((e_full + chunk - 1) // chunk) * chunk
    pad = e_pad - e_full
    rowf = jnp.concatenate([row, loop_idx, jnp.zeros((pad,), jnp.int32)])
    colf = jnp.concatenate([col, loop_idx, jnp.zeros((pad,), jnp.int32)])
    wf = jnp.concatenate([edge_weight.astype(jnp.float32),
                          jnp.ones((n,), jnp.float32),
                          jnp.zeros((pad,), jnp.float32)])

    degp = _deg_partials(colf, wf, n, e_pad)
    dis = _dis_kernel(degp)
    normf = _norm_kernel(dis, rowf, colf, wf, e_pad)

    h = _matmul(x, W1)
    p = _prop_kernel(h, rowf, colf, normf, e_pad)
    h = _ln_kernel(p, b1.reshape(1, d))
    h = _matmul(h, W2)
    p = _prop_kernel(h, rowf, colf, normf, e_pad)
    h = _ln_kernel(p, b2.reshape(1, d))
    return h


# speedup vs baseline: 10.1778x; 1.2310x over previous
import functools

import jax
import jax.numpy as jnp
from jax import lax
from jax.experimental import pallas as pl
from jax.experimental.pallas import tpu as pltpu
from jax.experimental.pallas import tpu_sc as plsc

NC = 2
NS = 16
L = 16
NTILES = NC * NS
BLK = 128
RING = 3
LN_EPS = 1e-5

_MESH = plsc.VectorSubcoreMesh(core_axis_name="c", subcore_axis_name="s")
_SC_PARAMS = pltpu.CompilerParams(needs_layout_passes=False)


def _zero_fill(buf, nrows, ncols):
    z = jnp.zeros((L,), jnp.float32)

    @pl.loop(0, nrows)
    def _(j):
        for k in range(ncols // L):
            buf[j, pl.ds(k * L, L)] = z


def _copy_rows(src_buf, dst, r0, total):
    full, rem = divmod(total, BLK)
    for i in range(full):
        pltpu.sync_copy(src_buf, dst.at[pl.ds(r0 + i * BLK, BLK)])
    if rem:
        pltpu.sync_copy(src_buf.at[pl.ds(0, rem)], dst.at[pl.ds(r0 + full * BLK, rem)])


def _sub_rows(n, s):
    base = (n // NS) & ~7
    last = n - base * (NS - 1)
    return s * base, base, last


def _zero_acc(zsrc, acc, n, s):
    r0, base, last = _sub_rows(n, s)
    _copy_rows(zsrc, acc, r0, base)

    @pl.when(s == NS - 1)
    def _():
        _copy_rows(zsrc, acc, base * NS, last - base)


def _dump_acc(acc, out_hbm, c, n, s):
    r0, base, last = _sub_rows(n, s)
    pltpu.sync_copy(acc.at[pl.ds(r0, base)], out_hbm.at[c, pl.ds(r0, base)])

    @pl.when(s == NS - 1)
    def _():
        ex = base * NS
        pltpu.sync_copy(acc.at[pl.ds(ex, last - base)],
                        out_hbm.at[c, pl.ds(ex, last - base)])


def _deg_partials(colf, wf, n, e_pad):
    ept = e_pad // NTILES
    nblk = ept // BLK
    assert nblk % RING == 0

    @functools.partial(
        pl.kernel,
        out_type=jax.ShapeDtypeStruct((NC, n, L), jnp.float32),
        mesh=_MESH,
        compiler_params=_SC_PARAMS,
        scratch_types=[
            pltpu.VMEM((BLK, L), jnp.float32),
            pltpu.VMEM((BLK,), jnp.int32),
            pltpu.VMEM((BLK,), jnp.float32),
            pltpu.VMEM_SHARED((n, L), jnp.float32),
        ],
    )
    def k(colf_hbm, wf_hbm, out_hbm, sbuf, icol, wbuf, acc):
        c = lax.axis_index("c")
        s = lax.axis_index("s")
        tid = c * NS + s

        z16i = jnp.zeros((L,), jnp.int32)
        iota16 = lax.iota(jnp.int32, L)
        _zero_fill(sbuf, BLK, L)
        _zero_acc(sbuf, acc, n, s)
        plsc.subcore_barrier()

        @pl.loop(0, nblk)
        def _(q):
            e0 = tid * ept + q * BLK
            pltpu.sync_copy(colf_hbm.at[pl.ds(e0, BLK)], icol)
            pltpu.sync_copy(wf_hbm.at[pl.ds(e0, BLK)], wbuf)
            for g in range(BLK // L):
                plsc.store_scatter(sbuf, [g * L + iota16, z16i],
                                   wbuf[pl.ds(g * L, L)])
            pltpu.sync_copy(sbuf, acc.at[icol], add=True)

        plsc.subcore_barrier()
        _dump_acc(acc, out_hbm, c, n, s)

    return k(colf, wf)


def _dis_kernel(degp):
    n = degp.shape[1]

    def body(p_ref, o_ref):
        deg = jnp.sum(p_ref[...], axis=(0, 2))
        o_ref[...] = lax.rsqrt(deg)

    return pl.pallas_call(
        body,
        out_shape=jax.ShapeDtypeStruct((n,), jnp.float32),
    )(degp)


def _norm_kernel(dis, rowf, colf, wf, e_pad):
    n = dis.shape[0]
    ept = e_pad // NTILES

    @functools.partial(
        pl.kernel,
        out_type=jax.ShapeDtypeStruct((e_pad,), jnp.float32),
        mesh=_MESH,
        compiler_params=_SC_PARAMS,
        scratch_types=[
            pltpu.VMEM((n,), jnp.float32),
            pltpu.VMEM((ept,), jnp.int32),
            pltpu.VMEM((ept,), jnp.int32),
            pltpu.VMEM((ept,), jnp.float32),
            pltpu.VMEM((ept,), jnp.float32),
        ],
    )
    def k(dis_hbm, rowf_hbm, colf_hbm, wf_hbm, out_hbm,
          disv, row_all, col_all, w_all, norm_all):
        c = lax.axis_index("c")
        s = lax.axis_index("s")
        tid = c * NS + s
        e0 = tid * ept
        pltpu.sync_copy(dis_hbm, disv)
        pltpu.sync_copy(rowf_hbm.at[pl.ds(e0, ept)], row_all)
        pltpu.sync_copy(colf_hbm.at[pl.ds(e0, ept)], col_all)
        pltpu.sync_copy(wf_hbm.at[pl.ds(e0, ept)], w_all)

        @pl.loop(0, ept // L)
        def _(g):
            sl = pl.ds(g * L, L)
            a = plsc.load_gather(disv, [row_all[sl]])
            b = plsc.load_gather(disv, [col_all[sl]])
            norm_all[sl] = a * w_all[sl] * b

        pltpu.sync_copy(norm_all, out_hbm.at[pl.ds(e0, ept)])

    return k(dis, rowf, colf, wf)


def _prop_kernel_sync(h, dis, pack, n, e_pad):
    d = h.shape[1]
    ept = e_pad // NTILES
    nblk = ept // BLK

    @functools.partial(
        pl.kernel,
        out_type=jax.ShapeDtypeStruct((NC, n, d), jnp.float32),
        mesh=_MESH,
        compiler_params=_SC_PARAMS,
        scratch_types=[
            pltpu.VMEM((n,), jnp.float32),
            pltpu.VMEM((BLK, d), jnp.float32),
            pltpu.VMEM((3, BLK), jnp.int32),
            pltpu.VMEM((BLK,), jnp.float32),
            pltpu.VMEM_SHARED((n, d), jnp.float32),
        ],
    )
    def k(h_hbm, dis_hbm, pack_hbm, out_hbm, disv, gbuf, pk, nbuf, acc):
        c = lax.axis_index("c")
        s = lax.axis_index("s")
        tid = c * NS + s
        q0 = tid * nblk
        pltpu.sync_copy(dis_hbm, disv)
        _zero_fill(gbuf, BLK, d)
        _zero_acc(gbuf, acc, n, s)
        plsc.subcore_barrier()
        z16 = jnp.zeros((L,), jnp.int32)
        one16 = jnp.full((L,), 1, jnp.int32)
        two16 = jnp.full((L,), 2, jnp.int32)
        iota16 = lax.iota(jnp.int32, L)

        @pl.loop(0, nblk)
        def _(q):
            pltpu.sync_copy(pack_hbm.at[q0 + q], pk)
            pltpu.sync_copy(h_hbm.at[pk.at[0]], gbuf)
            for g in range(BLK // L):
                gi = g * L + iota16
                r16 = plsc.load_gather(pk, [z16, gi])
                c16 = plsc.load_gather(pk, [one16, gi])
                w16 = plsc.bitcast(plsc.load_gather(pk, [two16, gi]), jnp.float32)
                a = plsc.load_gather(disv, [r16])
                bb = plsc.load_gather(disv, [c16])
                nbuf[pl.ds(g * L, L)] = a * w16 * bb

            @pl.loop(0, BLK)
            def _(j):
                nsplat = plsc.load_gather(nbuf, [jnp.full((L,), j, jnp.int32)])
                for k8 in range(d // L):
                    sl = pl.ds(k8 * L, L)
                    gbuf[j, sl] = gbuf[j, sl] * nsplat

            pltpu.sync_copy(gbuf, acc.at[pk.at[1]], add=True)

        plsc.subcore_barrier()
        _dump_acc(acc, out_hbm, c, n, s)

    return k(h, dis, pack)


def _prop_kernel(h, pack, n, e_pad):
    d = h.shape[1]
    ept = e_pad // NTILES
    nblk = ept // BLK
    assert nblk % RING == 0 and nblk // RING >= 2

    @functools.partial(
        pl.kernel,
        out_type=jax.ShapeDtypeStruct((NC, n, d), jnp.float32),
        mesh=_MESH,
        compiler_params=_SC_PARAMS,
        scratch_types=[
            pltpu.VMEM((BLK, d), jnp.float32),
            pltpu.VMEM((BLK, d), jnp.float32),
            pltpu.VMEM((BLK, d), jnp.float32),
            pltpu.VMEM((3, BLK), jnp.int32),
            pltpu.VMEM((3, BLK), jnp.int32),
            pltpu.VMEM((3, BLK), jnp.int32),
            pltpu.SemaphoreType.DMA,
            pltpu.SemaphoreType.DMA,
            pltpu.SemaphoreType.DMA,
            pltpu.SemaphoreType.DMA,
            pltpu.SemaphoreType.DMA,
            pltpu.SemaphoreType.DMA,
            pltpu.SemaphoreType.DMA,
            pltpu.SemaphoreType.DMA,
            pltpu.SemaphoreType.DMA,
            pltpu.VMEM_SHARED((n, d), jnp.float32),
        ],
    )
    def k(h_hbm, pack_hbm, out_hbm, gb0, gb1, gb2, pk0, pk1, pk2,
          is0, is1, is2, gs0, gs1, gs2, ss0, ss1, ss2, acc):
        c = lax.axis_index("c")
        s = lax.axis_index("s")
        tid = c * NS + s
        q0 = tid * nblk
        gbufs = (gb0, gb1, gb2)
        pks = (pk0, pk1, pk2)
        isems = (is0, is1, is2)
        gsems = (gs0, gs1, gs2)
        ssems = (ss0, ss1, ss2)

        _zero_fill(gbufs[0], BLK, d)
        _zero_acc(gbufs[0], acc, n, s)
        plsc.subcore_barrier()

        def issue_idx(q, b):
            pltpu.async_copy(pack_hbm.at[q0 + q], pks[b], isems[b])

        def wait_idx(q, b):
            pltpu.make_async_copy(pack_hbm.at[q0 + q], pks[b], isems[b]).wait()

        def issue_gather(b):
            pltpu.async_copy(h_hbm.at[pks[b].at[0]], gbufs[b], gsems[b])

        def wait_gather(b):
            pltpu.make_async_copy(h_hbm.at[pks[b].at[0]], gbufs[b],
                                  gsems[b]).wait()

        def issue_scatter(b):
            pltpu.async_copy(gbufs[b], acc.at[pks[b].at[1]], ssems[b], add=True)

        def wait_scatter(b):
            pltpu.make_async_copy(gbufs[b], acc.at[pks[b].at[1]],
                                  ssems[b]).wait()

        two16 = jnp.full((L,), 2, jnp.int32)

        def scale(b):
            @pl.loop(0, BLK)
            def _(j):
                ni = plsc.load_gather(pks[b], [two16, jnp.full((L,), j, jnp.int32)])
                nsplat = plsc.bitcast(ni, jnp.float32)
                for k8 in range(d // L):
                    sl = pl.ds(k8 * L, L)
                    gbufs[b][j, sl] = gbufs[b][j, sl] * nsplat

        def do_block(q, b, bn1, bn2, first, last, pf_idx):
            if not last:
                wait_idx(q + 1, bn1)
                issue_gather(bn1)
            wait_gather(b)
            scale(b)
            issue_scatter(b)
            if not first:
                wait_scatter(bn2)
            if pf_idx:
                issue_idx(q + 2, bn2)

        issue_idx(0, 0)
        issue_idx(1, 1)
        wait_idx(0, 0)
        issue_gather(0)
        do_block(0, 0, 1, 2, True, False, True)
        do_block(1, 1, 2, 0, False, False, True)
        do_block(2, 2, 0, 1, False, False, True)

        @pl.loop(1, nblk // RING - 1)
        def _(grp):
            q = grp * RING
            do_block(q, 0, 1, 2, False, False, True)
            do_block(q + 1, 1, 2, 0, False, False, True)
            do_block(q + 2, 2, 0, 1, False, False, True)

        do_block(nblk - 3, 0, 1, 2, False, False, True)
        do_block(nblk - 2, 1, 2, 0, False, False, False)
        do_block(nblk - 1, 2, 0, 1, False, True, False)

        wait_scatter(2)
        plsc.subcore_barrier()
        _dump_acc(acc, out_hbm, c, n, s)

    return k(h, pack)


def _matmul(x, w):
    n, d = x.shape
    blk = 1000

    def body(x_ref, w_ref, o_ref):
        o_ref[...] = jnp.dot(x_ref[...], w_ref[...],
                             preferred_element_type=jnp.float32)

    return pl.pallas_call(
        body,
        out_shape=jax.ShapeDtypeStruct((n, d), jnp.float32),
        grid=(n // blk,),
        in_specs=[
            pl.BlockSpec((blk, d), lambda i: (i, 0)),
            pl.BlockSpec((d, d), lambda i: (0, 0)),
        ],
        out_specs=pl.BlockSpec((blk, d), lambda i: (i, 0)),
    )(x, w)


def _ln_kernel(p, b):
    _, n, d = p.shape
    blk = 1000

    def body(p_ref, b_ref, o_ref):
        t = p_ref[0] + p_ref[1] + b_ref[...]
        mu = jnp.mean(t, axis=-1, keepdims=True)
        var = jnp.mean((t - mu) ** 2, axis=-1, keepdims=True)
        y = (t - mu) * lax.rsqrt(var + LN_EPS)
        o_ref[...] = jnp.maximum(y, 0.0)

    return pl.pallas_call(
        body,
        out_shape=jax.ShapeDtypeStruct((n, d), jnp.float32),
        grid=(n // blk,),
        in_specs=[
            pl.BlockSpec((2, blk, d), lambda i: (0, i, 0)),
            pl.BlockSpec((1, d), lambda i: (0, 0)),
        ],
        out_specs=pl.BlockSpec((blk, d), lambda i: (i, 0)),
    )(p, b)


def kernel(x, edge_index, edge_weight, W1, b1, W2, b2):
    n, d = x.shape
    e = edge_weight.shape[0]
    row = edge_index[0].astype(jnp.int32)
    col = edge_index[1].astype(jnp.int32)
    loop_idx = jnp.arange(n, dtype=jnp.int32)
    e_full = e + n
    chunk = NTILES * BLK * RING
    e_pad = ((e_full + chunk - 1) // chunk) * chunk
    pad = e_pad - e_full
    rowf = jnp.concatenate([row, loop_idx, jnp.zeros((pad,), jnp.int32)])
    colf = jnp.concatenate([col, loop_idx, jnp.zeros((pad,), jnp.int32)])
    wf = jnp.concatenate([edge_weight.astype(jnp.float32),
                          jnp.ones((n,), jnp.float32),
                          jnp.zeros((pad,), jnp.float32)])

    degp = _deg_partials(colf, wf, n, e_pad)
    dis = _dis_kernel(degp)

    nb_tot = e_pad // BLK
    pack = jnp.stack([rowf.reshape(nb_tot, BLK),
                      colf.reshape(nb_tot, BLK),
                      lax.bitcast_convert_type(wf, jnp.int32)
                         .reshape(nb_tot, BLK)], axis=1)

    h = _matmul(x, W1)
    p = _prop_kernel_sync(h, dis, pack, n, e_pad)
    h = _ln_kernel(p, b1.reshape(1, d))
    h = _matmul(h, W2)
    p = _prop_kernel_sync(h, dis, pack, n, e_pad)
    h = _ln_kernel(p, b2.reshape(1, d))
    return h


# speedup vs baseline: 14.6340x; 1.4378x over previous
import functools

import jax
import jax.numpy as jnp
from jax import lax
from jax.experimental import pallas as pl
from jax.experimental.pallas import tpu as pltpu
from jax.experimental.pallas import tpu_sc as plsc

NC = 2
NS = 16
L = 16
NTILES = NC * NS
BLK = 128
RING = 3
LN_EPS = 1e-5

_MESH = plsc.VectorSubcoreMesh(core_axis_name="c", subcore_axis_name="s")
_SC_PARAMS = pltpu.CompilerParams(needs_layout_passes=False)


def _zero_fill(buf, nrows, ncols):
    z = jnp.zeros((L,), jnp.float32)

    @pl.loop(0, nrows)
    def _(j):
        for k in range(ncols // L):
            buf[j, pl.ds(k * L, L)] = z


def _copy_rows(src_buf, dst, r0, total):
    full, rem = divmod(total, BLK)
    for i in range(full):
        pltpu.sync_copy(src_buf, dst.at[pl.ds(r0 + i * BLK, BLK)])
    if rem:
        pltpu.sync_copy(src_buf.at[pl.ds(0, rem)], dst.at[pl.ds(r0 + full * BLK, rem)])


def _sub_rows(n, s):
    base = (n // NS) & ~7
    last = n - base * (NS - 1)
    return s * base, base, last


def _zero_acc(zsrc, acc, n, s):
    r0, base, last = _sub_rows(n, s)
    _copy_rows(zsrc, acc, r0, base)

    @pl.when(s == NS - 1)
    def _():
        _copy_rows(zsrc, acc, base * NS, last - base)


def _dump_acc(acc, out_hbm, c, n, s):
    r0, base, last = _sub_rows(n, s)
    pltpu.sync_copy(acc.at[pl.ds(r0, base)], out_hbm.at[c, pl.ds(r0, base)])

    @pl.when(s == NS - 1)
    def _():
        ex = base * NS
        pltpu.sync_copy(acc.at[pl.ds(ex, last - base)],
                        out_hbm.at[c, pl.ds(ex, last - base)])


def _deg_partials(colf, wf, n, e_pad):
    ept = e_pad // NTILES
    nblk = ept // BLK
    assert nblk % RING == 0

    @functools.partial(
        pl.kernel,
        out_type=jax.ShapeDtypeStruct((NC, n, L), jnp.float32),
        mesh=_MESH,
        compiler_params=_SC_PARAMS,
        scratch_types=[
            pltpu.VMEM((BLK, L), jnp.float32),
            pltpu.VMEM((BLK,), jnp.int32),
            pltpu.VMEM((BLK,), jnp.float32),
            pltpu.VMEM_SHARED((n, L), jnp.float32),
        ],
    )
    def k(colf_hbm, wf_hbm, out_hbm, sbuf, icol, wbuf, acc):
        c = lax.axis_index("c")
        s = lax.axis_index("s")
        tid = c * NS + s

        z16i = jnp.zeros((L,), jnp.int32)
        iota16 = lax.iota(jnp.int32, L)
        _zero_fill(sbuf, BLK, L)
        _zero_acc(sbuf, acc, n, s)
        plsc.subcore_barrier()

        @pl.loop(0, nblk)
        def _(q):
            e0 = tid * ept + q * BLK
            pltpu.sync_copy(colf_hbm.at[pl.ds(e0, BLK)], icol)
            pltpu.sync_copy(wf_hbm.at[pl.ds(e0, BLK)], wbuf)
            for g in range(BLK // L):
                plsc.store_scatter(sbuf, [g * L + iota16, z16i],
                                   wbuf[pl.ds(g * L, L)])
            pltpu.sync_copy(sbuf, acc.at[icol], add=True)

        plsc.subcore_barrier()
        _dump_acc(acc, out_hbm, c, n, s)

    return k(colf, wf)


def _dis_kernel(degp):
    n = degp.shape[1]

    def body(p_ref, o_ref):
        deg = jnp.sum(p_ref[...], axis=(0, 2))
        o_ref[...] = lax.rsqrt(deg)

    return pl.pallas_call(
        body,
        out_shape=jax.ShapeDtypeStruct((n,), jnp.float32),
    )(degp)


def _norm_kernel(dis, rowf, colf, wf, e_pad):
    n = dis.shape[0]
    ept = e_pad // NTILES

    @functools.partial(
        pl.kernel,
        out_type=jax.ShapeDtypeStruct((e_pad,), jnp.float32),
        mesh=_MESH,
        compiler_params=_SC_PARAMS,
        scratch_types=[
            pltpu.VMEM((n,), jnp.float32),
            pltpu.VMEM((ept,), jnp.int32),
            pltpu.VMEM((ept,), jnp.int32),
            pltpu.VMEM((ept,), jnp.float32),
            pltpu.VMEM((ept,), jnp.float32),
        ],
    )
    def k(dis_hbm, rowf_hbm, colf_hbm, wf_hbm, out_hbm,
          disv, row_all, col_all, w_all, norm_all):
        c = lax.axis_index("c")
        s = lax.axis_index("s")
        tid = c * NS + s
        e0 = tid * ept
        pltpu.sync_copy(dis_hbm, disv)
        pltpu.sync_copy(rowf_hbm.at[pl.ds(e0, ept)], row_all)
        pltpu.sync_copy(colf_hbm.at[pl.ds(e0, ept)], col_all)
        pltpu.sync_copy(wf_hbm.at[pl.ds(e0, ept)], w_all)

        @pl.loop(0, ept // L)
        def _(g):
            sl = pl.ds(g * L, L)
            a = plsc.load_gather(disv, [row_all[sl]])
            b = plsc.load_gather(disv, [col_all[sl]])
            norm_all[sl] = a * w_all[sl] * b

        pltpu.sync_copy(norm_all, out_hbm.at[pl.ds(e0, ept)])

    return k(dis, rowf, colf, wf)


def _norm_pipe(dis, pack, e_pad):
    n = dis.shape[0]
    ept = e_pad // NTILES
    nblk = ept // BLK
    assert nblk % RING == 0 and nblk // RING >= 2

    @functools.partial(
        pl.kernel,
        out_type=jax.ShapeDtypeStruct((e_pad,), jnp.float32),
        mesh=_MESH,
        compiler_params=_SC_PARAMS,
        scratch_types=[
            pltpu.VMEM((n,), jnp.float32),
            pltpu.VMEM((3, BLK), jnp.int32),
            pltpu.VMEM((3, BLK), jnp.int32),
            pltpu.VMEM((3, BLK), jnp.int32),
            pltpu.VMEM((BLK,), jnp.float32),
            pltpu.VMEM((BLK,), jnp.float32),
            pltpu.VMEM((BLK,), jnp.float32),
            pltpu.SemaphoreType.DMA,
            pltpu.SemaphoreType.DMA,
            pltpu.SemaphoreType.DMA,
            pltpu.SemaphoreType.DMA,
            pltpu.SemaphoreType.DMA,
            pltpu.SemaphoreType.DMA,
        ],
    )
    def k(dis_hbm, pack_hbm, out_hbm, disv, pk0, pk1, pk2, nb0, nb1, nb2,
          is0, is1, is2, os0, os1, os2):
        c = lax.axis_index("c")
        s = lax.axis_index("s")
        tid = c * NS + s
        q0 = tid * nblk
        e0 = tid * ept
        pks = (pk0, pk1, pk2)
        nbufs = (nb0, nb1, nb2)
        isems = (is0, is1, is2)
        osems = (os0, os1, os2)
        pltpu.sync_copy(dis_hbm, disv)

        z16 = jnp.zeros((L,), jnp.int32)
        one16 = jnp.full((L,), 1, jnp.int32)
        two16 = jnp.full((L,), 2, jnp.int32)
        iota16 = lax.iota(jnp.int32, L)

        def issue_idx(q, b):
            pltpu.async_copy(pack_hbm.at[q0 + q], pks[b], isems[b])

        def wait_idx(q, b):
            pltpu.make_async_copy(pack_hbm.at[q0 + q], pks[b], isems[b]).wait()

        def issue_out(q, b):
            pltpu.async_copy(nbufs[b], out_hbm.at[pl.ds(e0 + q * BLK, BLK)],
                             osems[b])

        def wait_out(q, b):
            pltpu.make_async_copy(nbufs[b],
                                  out_hbm.at[pl.ds(e0 + q * BLK, BLK)],
                                  osems[b]).wait()

        def do_block(q, b, first, pf_idx):
            wait_idx(q, b)
            if pf_idx:
                issue_idx(q + 2, (b + 2) % RING)
            if not first:
                wait_out(q - RING, b)
            for g in range(BLK // L):
                gi = g * L + iota16
                r16 = plsc.load_gather(pks[b], [z16, gi])
                c16 = plsc.load_gather(pks[b], [one16, gi])
                w16 = plsc.bitcast(plsc.load_gather(pks[b], [two16, gi]),
                                   jnp.float32)
                a = plsc.load_gather(disv, [r16])
                bb = plsc.load_gather(disv, [c16])
                nbufs[b][pl.ds(g * L, L)] = a * w16 * bb
            issue_out(q, b)

        issue_idx(0, 0)
        issue_idx(1, 1)
        do_block(0, 0, True, True)
        do_block(1, 1, True, True)
        do_block(2, 2, True, True)

        @pl.loop(1, nblk // RING - 1)
        def _(grp):
            q = grp * RING
            do_block(q, 0, False, True)
            do_block(q + 1, 1, False, True)
            do_block(q + 2, 2, False, True)

        do_block(nblk - 3, 0, False, True)
        do_block(nblk - 2, 1, False, False)
        do_block(nblk - 1, 2, False, False)

        wait_out(nblk - 3, 0)
        wait_out(nblk - 2, 1)
        wait_out(nblk - 1, 2)

    return k(dis, pack)


def _prop_kernel_sync(h, dis, pack, n, e_pad):
    d = h.shape[1]
    ept = e_pad // NTILES
    nblk = ept // BLK

    @functools.partial(
        pl.kernel,
        out_type=jax.ShapeDtypeStruct((NC, n, d), jnp.float32),
        mesh=_MESH,
        compiler_params=_SC_PARAMS,
        scratch_types=[
            pltpu.VMEM((n,), jnp.float32),
            pltpu.VMEM((BLK, d), jnp.float32),
            pltpu.VMEM((3, BLK), jnp.int32),
            pltpu.VMEM((BLK,), jnp.float32),
            pltpu.VMEM_SHARED((n, d), jnp.float32),
        ],
    )
    def k(h_hbm, dis_hbm, pack_hbm, out_hbm, disv, gbuf, pk, nbuf, acc):
        c = lax.axis_index("c")
        s = lax.axis_index("s")
        tid = c * NS + s
        q0 = tid * nblk
        pltpu.sync_copy(dis_hbm, disv)
        _zero_fill(gbuf, BLK, d)
        _zero_acc(gbuf, acc, n, s)
        plsc.subcore_barrier()
        z16 = jnp.zeros((L,), jnp.int32)
        one16 = jnp.full((L,), 1, jnp.int32)
        two16 = jnp.full((L,), 2, jnp.int32)
        iota16 = lax.iota(jnp.int32, L)

        @pl.loop(0, nblk)
        def _(q):
            pltpu.sync_copy(pack_hbm.at[q0 + q], pk)
            pltpu.sync_copy(h_hbm.at[pk.at[0]], gbuf)
            for g in range(BLK // L):
                gi = g * L + iota16
                r16 = plsc.load_gather(pk, [z16, gi])
                c16 = plsc.load_gather(pk, [one16, gi])
                w16 = plsc.bitcast(plsc.load_gather(pk, [two16, gi]), jnp.float32)
                a = plsc.load_gather(disv, [r16])
                bb = plsc.load_gather(disv, [c16])
                nbuf[pl.ds(g * L, L)] = a * w16 * bb

            @pl.loop(0, BLK)
            def _(j):
                nsplat = plsc.load_gather(nbuf, [jnp.full((L,), j, jnp.int32)])
                for k8 in range(d // L):
                    sl = pl.ds(k8 * L, L)
                    gbuf[j, sl] = gbuf[j, sl] * nsplat

            pltpu.sync_copy(gbuf, acc.at[pk.at[1]], add=True)

        plsc.subcore_barrier()
        _dump_acc(acc, out_hbm, c, n, s)

    return k(h, dis, pack)


def _prop_kernel(h, pack, n, e_pad):
    d = h.shape[1]
    ept = e_pad // NTILES
    nblk = ept // BLK
    assert nblk % RING == 0 and nblk // RING >= 2

    @functools.partial(
        pl.kernel,
        out_type=jax.ShapeDtypeStruct((NC, n, d), jnp.float32),
        mesh=_MESH,
        compiler_params=_SC_PARAMS,
        scratch_types=[
            pltpu.VMEM((BLK, d), jnp.float32),
            pltpu.VMEM((BLK, d), jnp.float32),
            pltpu.VMEM((BLK, d), jnp.float32),
            pltpu.VMEM((3, BLK), jnp.int32),
            pltpu.VMEM((3, BLK), jnp.int32),
            pltpu.VMEM((3, BLK), jnp.int32),
            pltpu.SemaphoreType.DMA,
            pltpu.SemaphoreType.DMA,
            pltpu.SemaphoreType.DMA,
            pltpu.SemaphoreType.DMA,
            pltpu.SemaphoreType.DMA,
            pltpu.SemaphoreType.DMA,
            pltpu.SemaphoreType.DMA,
            pltpu.SemaphoreType.DMA,
            pltpu.SemaphoreType.DMA,
            pltpu.VMEM_SHARED((n, d), jnp.float32),
        ],
    )
    def k(h_hbm, pack_hbm, out_hbm, gb0, gb1, gb2, pk0, pk1, pk2,
          is0, is1, is2, gs0, gs1, gs2, ss0, ss1, ss2, acc):
        c = lax.axis_index("c")
        s = lax.axis_index("s")
        tid = c * NS + s
        q0 = tid * nblk
        gbufs = (gb0, gb1, gb2)
        pks = (pk0, pk1, pk2)
        isems = (is0, is1, is2)
        gsems = (gs0, gs1, gs2)
        ssems = (ss0, ss1, ss2)

        _zero_fill(gbufs[0], BLK, d)
        _zero_acc(gbufs[0], acc, n, s)
        plsc.subcore_barrier()

        def issue_idx(q, b):
            pltpu.async_copy(pack_hbm.at[q0 + q], pks[b], isems[b])

        def wait_idx(q, b):
            pltpu.make_async_copy(pack_hbm.at[q0 + q], pks[b], isems[b]).wait()

        def issue_gather(b):
            pltpu.async_copy(h_hbm.at[pks[b].at[0]], gbufs[b], gsems[b])

        def wait_gather(b):
            pltpu.make_async_copy(h_hbm.at[pks[b].at[0]], gbufs[b],
                                  gsems[b]).wait()

        def issue_scatter(b):
            pltpu.async_copy(gbufs[b], acc.at[pks[b].at[1]], ssems[b], add=True)

        def wait_scatter(b):
            pltpu.make_async_copy(gbufs[b], acc.at[pks[b].at[1]],
                                  ssems[b]).wait()

        two16 = jnp.full((L,), 2, jnp.int32)

        def scale(b):
            @pl.loop(0, BLK)
            def _(j):
                ni = plsc.load_gather(pks[b], [two16, jnp.full((L,), j, jnp.int32)])
                nsplat = plsc.bitcast(ni, jnp.float32)
                for k8 in range(d // L):
                    sl = pl.ds(k8 * L, L)
                    gbufs[b][j, sl] = gbufs[b][j, sl] * nsplat

        def do_block(q, b, bn1, bn2, first, last, pf_idx):
            if not last:
                wait_idx(q + 1, bn1)
                issue_gather(bn1)
            wait_gather(b)
            scale(b)
            issue_scatter(b)
            if not first:
                wait_scatter(bn2)
            if pf_idx:
                issue_idx(q + 2, bn2)

        issue_idx(0, 0)
        issue_idx(1, 1)
        wait_idx(0, 0)
        issue_gather(0)
        do_block(0, 0, 1, 2, True, False, True)
        do_block(1, 1, 2, 0, False, False, True)
        do_block(2, 2, 0, 1, False, False, True)

        @pl.loop(1, nblk // RING - 1)
        def _(grp):
            q = grp * RING
            do_block(q, 0, 1, 2, False, False, True)
            do_block(q + 1, 1, 2, 0, False, False, True)
            do_block(q + 2, 2, 0, 1, False, False, True)

        do_block(nblk - 3, 0, 1, 2, False, False, True)
        do_block(nblk - 2, 1, 2, 0, False, False, False)
        do_block(nblk - 1, 2, 0, 1, False, True, False)

        wait_scatter(2)
        plsc.subcore_barrier()
        _dump_acc(acc, out_hbm, c, n, s)

    return k(h, pack)


def _matmul(x, w):
    n, d = x.shape
    blk = 1000

    def body(x_ref, w_ref, o_ref):
        o_ref[...] = jnp.dot(x_ref[...], w_ref[...],
                             preferred_element_type=jnp.float32)

    return pl.pallas_call(
        body,
        out_shape=jax.ShapeDtypeStruct((n, d), jnp.float32),
        grid=(n // blk,),
        in_specs=[
            pl.BlockSpec((blk, d), lambda i: (i, 0)),
            pl.BlockSpec((d, d), lambda i: (0, 0)),
        ],
        out_specs=pl.BlockSpec((blk, d), lambda i: (i, 0)),
    )(x, w)


def _ln_kernel(p, b):
    _, n, d = p.shape
    blk = 1000

    def body(p_ref, b_ref, o_ref):
        t = p_ref[0] + p_ref[1] + b_ref[...]
        mu = jnp.mean(t, axis=-1, keepdims=True)
        var = jnp.mean((t - mu) ** 2, axis=-1, keepdims=True)
        y = (t - mu) * lax.rsqrt(var + LN_EPS)
        o_ref[...] = jnp.maximum(y, 0.0)

    return pl.pallas_call(
        body,
        out_shape=jax.ShapeDtypeStruct((n, d), jnp.float32),
        grid=(n // blk,),
        in_specs=[
            pl.BlockSpec((2, blk, d), lambda i: (0, i, 0)),
            pl.BlockSpec((1, d), lambda i: (0, 0)),
        ],
        out_specs=pl.BlockSpec((blk, d), lambda i: (i, 0)),
    )(p, b)


def kernel(x, edge_index, edge_weight, W1, b1, W2, b2):
    n, d = x.shape
    e = edge_weight.shape[0]
    row = edge_index[0].astype(jnp.int32)
    col = edge_index[1].astype(jnp.int32)
    loop_idx = jnp.arange(n, dtype=jnp.int32)
    e_full = e + n
    chunk = NTILES * BLK * RING
    e_pad = ((e_full + chunk - 1) // chunk) * chunk
    pad = e_pad - e_full
    rowf = jnp.concatenate([row, loop_idx, jnp.zeros((pad,), jnp.int32)])
    colf = jnp.concatenate([col, loop_idx, jnp.zeros((pad,), jnp.int32)])
    wf = jnp.concatenate([edge_weight.astype(jnp.float32),
                          jnp.ones((n,), jnp.float32),
                          jnp.zeros((pad,), jnp.float32)])

    degp = _deg_partials(colf, wf, n, e_pad)
    dis = _dis_kernel(degp)

    nb_tot = e_pad // BLK
    rowb = rowf.reshape(nb_tot, BLK)
    colb = colf.reshape(nb_tot, BLK)
    pack1 = jnp.stack([rowb, colb,
                       lax.bitcast_convert_type(wf, jnp.int32)
                          .reshape(nb_tot, BLK)], axis=1)
    normf = _norm_pipe(dis, pack1, e_pad)
    pack2 = jnp.stack([rowb, colb,
                       lax.bitcast_convert_type(normf, jnp.int32)
                          .reshape(nb_tot, BLK)], axis=1)

    h = _matmul(x, W1)
    p = _prop_kernel(h, pack2, n, e_pad)
    h = _ln_kernel(p, b1.reshape(1, d))
    h = _matmul(h, W2)
    p = _prop_kernel(h, pack2, n, e_pad)
    h = _ln_kernel(p, b2.reshape(1, d))
    return h


# speedup vs baseline: 15.4631x; 1.0567x over previous
import functools

import jax
import jax.numpy as jnp
from jax import lax
from jax.experimental import pallas as pl
from jax.experimental.pallas import tpu as pltpu
from jax.experimental.pallas import tpu_sc as plsc

NC = 2
NS = 16
L = 16
NTILES = NC * NS
BLK = 128
RING = 3
LN_EPS = 1e-5

_MESH = plsc.VectorSubcoreMesh(core_axis_name="c", subcore_axis_name="s")
_SC_PARAMS = pltpu.CompilerParams(needs_layout_passes=False)


def _zero_fill(buf, nrows, ncols):
    z = jnp.zeros((L,), jnp.float32)

    @pl.loop(0, nrows)
    def _(j):
        for k in range(ncols // L):
            buf[j, pl.ds(k * L, L)] = z


def _copy_rows(src_buf, dst, r0, total):
    full, rem = divmod(total, BLK)
    for i in range(full):
        pltpu.sync_copy(src_buf, dst.at[pl.ds(r0 + i * BLK, BLK)])
    if rem:
        pltpu.sync_copy(src_buf.at[pl.ds(0, rem)], dst.at[pl.ds(r0 + full * BLK, rem)])


def _sub_rows(n, s):
    base = (n // NS) & ~7
    last = n - base * (NS - 1)
    return s * base, base, last


def _zero_acc(zsrc, acc, n, s):
    r0, base, last = _sub_rows(n, s)
    _copy_rows(zsrc, acc, r0, base)

    @pl.when(s == NS - 1)
    def _():
        _copy_rows(zsrc, acc, base * NS, last - base)


def _dump_acc(acc, out_hbm, c, n, s):
    r0, base, last = _sub_rows(n, s)
    pltpu.sync_copy(acc.at[pl.ds(r0, base)], out_hbm.at[c, pl.ds(r0, base)])

    @pl.when(s == NS - 1)
    def _():
        ex = base * NS
        pltpu.sync_copy(acc.at[pl.ds(ex, last - base)],
                        out_hbm.at[c, pl.ds(ex, last - base)])


def _deg_partials(colf, wf, n, e_pad):
    ept = e_pad // NTILES
    nblk = ept // BLK
    assert nblk % RING == 0

    @functools.partial(
        pl.kernel,
        out_type=jax.ShapeDtypeStruct((NC, n, L), jnp.float32),
        mesh=_MESH,
        compiler_params=_SC_PARAMS,
        scratch_types=[
            pltpu.VMEM((BLK, L), jnp.float32),
            pltpu.VMEM((BLK,), jnp.int32),
            pltpu.VMEM((BLK,), jnp.float32),
            pltpu.VMEM_SHARED((n, L), jnp.float32),
        ],
    )
    def k(colf_hbm, wf_hbm, out_hbm, sbuf, icol, wbuf, acc):
        c = lax.axis_index("c")
        s = lax.axis_index("s")
        tid = c * NS + s

        z16i = jnp.zeros((L,), jnp.int32)
        iota16 = lax.iota(jnp.int32, L)
        _zero_fill(sbuf, BLK, L)
        _zero_acc(sbuf, acc, n, s)
        plsc.subcore_barrier()

        @pl.loop(0, nblk)
        def _(q):
            e0 = tid * ept + q * BLK
            pltpu.sync_copy(colf_hbm.at[pl.ds(e0, BLK)], icol)
            pltpu.sync_copy(wf_hbm.at[pl.ds(e0, BLK)], wbuf)
            for g in range(BLK // L):
                plsc.store_scatter(sbuf, [g * L + iota16, z16i],
                                   wbuf[pl.ds(g * L, L)])
            pltpu.sync_copy(sbuf, acc.at[icol], add=True)

        plsc.subcore_barrier()
        _dump_acc(acc, out_hbm, c, n, s)

    return k(colf, wf)


def _deg_pipe(pack, n, e_pad):
    ept = e_pad // NTILES
    nblk = ept // BLK
    assert nblk % RING == 0 and nblk // RING >= 2

    @functools.partial(
        pl.kernel,
        out_type=jax.ShapeDtypeStruct((NC, n, L), jnp.float32),
        mesh=_MESH,
        compiler_params=_SC_PARAMS,
        scratch_types=[
            pltpu.VMEM((3, BLK), jnp.int32),
            pltpu.VMEM((3, BLK), jnp.int32),
            pltpu.VMEM((3, BLK), jnp.int32),
            pltpu.VMEM((BLK, L), jnp.float32),
            pltpu.VMEM((BLK, L), jnp.float32),
            pltpu.VMEM((BLK, L), jnp.float32),
            pltpu.VMEM((BLK,), jnp.int32),
            pltpu.VMEM((BLK,), jnp.int32),
            pltpu.VMEM((BLK,), jnp.int32),
            pltpu.VMEM_SHARED((n, L), jnp.float32),
            pltpu.SemaphoreType.DMA,
            pltpu.SemaphoreType.DMA,
            pltpu.SemaphoreType.DMA,
            pltpu.SemaphoreType.DMA,
            pltpu.SemaphoreType.DMA,
            pltpu.SemaphoreType.DMA,
        ],
    )
    def k(pack_hbm, out_hbm, pk0, pk1, pk2, sb0, sb1, sb2, ic0, ic1, ic2,
          acc, is0, is1, is2, ss0, ss1, ss2):
        c = lax.axis_index("c")
        s = lax.axis_index("s")
        tid = c * NS + s
        q0 = tid * nblk
        pks = (pk0, pk1, pk2)
        sbufs = (sb0, sb1, sb2)
        icbs = (ic0, ic1, ic2)
        isems = (is0, is1, is2)
        ssems = (ss0, ss1, ss2)

        one16 = jnp.full((L,), 1, jnp.int32)
        two16 = jnp.full((L,), 2, jnp.int32)
        iota16 = lax.iota(jnp.int32, L)
        z16i = jnp.zeros((L,), jnp.int32)
        for b in range(RING):
            _zero_fill(sbufs[b], BLK, L)
        _zero_acc(sbufs[0], acc, n, s)
        plsc.subcore_barrier()

        def issue_idx(q, b):
            pltpu.async_copy(pack_hbm.at[q0 + q], pks[b], isems[b])

        def wait_idx(q, b):
            pltpu.make_async_copy(pack_hbm.at[q0 + q], pks[b], isems[b]).wait()

        def issue_scatter(b):
            pltpu.async_copy(sbufs[b], acc.at[icbs[b]], ssems[b], add=True)

        def wait_scatter(b):
            pltpu.make_async_copy(sbufs[b], acc.at[icbs[b]], ssems[b]).wait()

        def do_block(q, b, first, pf_idx):
            wait_idx(q, b)
            if pf_idx:
                issue_idx(q + 2, (b + 2) % RING)
            if not first:
                wait_scatter(b)
            for g in range(BLK // L):
                gi = g * L + iota16
                icbs[b][pl.ds(g * L, L)] = plsc.load_gather(pks[b], [one16, gi])
                w16 = plsc.bitcast(plsc.load_gather(pks[b], [two16, gi]),
                                   jnp.float32)
                plsc.store_scatter(sbufs[b], [gi, z16i], w16)
            issue_scatter(b)

        issue_idx(0, 0)
        issue_idx(1, 1)
        do_block(0, 0, True, True)
        do_block(1, 1, True, True)
        do_block(2, 2, True, True)

        @pl.loop(1, nblk // RING - 1)
        def _(grp):
            q = grp * RING
            do_block(q, 0, False, True)
            do_block(q + 1, 1, False, True)
            do_block(q + 2, 2, False, True)

        do_block(nblk - 3, 0, False, True)
        do_block(nblk - 2, 1, False, False)
        do_block(nblk - 1, 2, False, False)

        for b in range(RING):
            wait_scatter(b)
        plsc.subcore_barrier()
        _dump_acc(acc, out_hbm, c, n, s)

    return k(pack)


def _dis_kernel(degp):
    n = degp.shape[1]

    def body(p_ref, o_ref):
        deg = jnp.sum(p_ref[...], axis=(0, 2))
        o_ref[...] = lax.rsqrt(deg)

    return pl.pallas_call(
        body,
        out_shape=jax.ShapeDtypeStruct((n,), jnp.float32),
    )(degp)


def _norm_kernel(dis, rowf, colf, wf, e_pad):
    n = dis.shape[0]
    ept = e_pad // NTILES

    @functools.partial(
        pl.kernel,
        out_type=jax.ShapeDtypeStruct((e_pad,), jnp.float32),
        mesh=_MESH,
        compiler_params=_SC_PARAMS,
        scratch_types=[
            pltpu.VMEM((n,), jnp.float32),
            pltpu.VMEM((ept,), jnp.int32),
            pltpu.VMEM((ept,), jnp.int32),
            pltpu.VMEM((ept,), jnp.float32),
            pltpu.VMEM((ept,), jnp.float32),
        ],
    )
    def k(dis_hbm, rowf_hbm, colf_hbm, wf_hbm, out_hbm,
          disv, row_all, col_all, w_all, norm_all):
        c = lax.axis_index("c")
        s = lax.axis_index("s")
        tid = c * NS + s
        e0 = tid * ept
        pltpu.sync_copy(dis_hbm, disv)
        pltpu.sync_copy(rowf_hbm.at[pl.ds(e0, ept)], row_all)
        pltpu.sync_copy(colf_hbm.at[pl.ds(e0, ept)], col_all)
        pltpu.sync_copy(wf_hbm.at[pl.ds(e0, ept)], w_all)

        @pl.loop(0, ept // L)
        def _(g):
            sl = pl.ds(g * L, L)
            a = plsc.load_gather(disv, [row_all[sl]])
            b = plsc.load_gather(disv, [col_all[sl]])
            norm_all[sl] = a * w_all[sl] * b

        pltpu.sync_copy(norm_all, out_hbm.at[pl.ds(e0, ept)])

    return k(dis, rowf, colf, wf)


def _norm_pipe(dis, pack, e_pad):
    n = dis.shape[0]
    ept = e_pad // NTILES
    nblk = ept // BLK
    assert nblk % RING == 0 and nblk // RING >= 2

    @functools.partial(
        pl.kernel,
        out_type=jax.ShapeDtypeStruct((e_pad,), jnp.float32),
        mesh=_MESH,
        compiler_params=_SC_PARAMS,
        scratch_types=[
            pltpu.VMEM((n,), jnp.float32),
            pltpu.VMEM((3, BLK), jnp.int32),
            pltpu.VMEM((3, BLK), jnp.int32),
            pltpu.VMEM((3, BLK), jnp.int32),
            pltpu.VMEM((BLK,), jnp.float32),
            pltpu.VMEM((BLK,), jnp.float32),
            pltpu.VMEM((BLK,), jnp.float32),
            pltpu.SemaphoreType.DMA,
            pltpu.SemaphoreType.DMA,
            pltpu.SemaphoreType.DMA,
            pltpu.SemaphoreType.DMA,
            pltpu.SemaphoreType.DMA,
            pltpu.SemaphoreType.DMA,
        ],
    )
    def k(dis_hbm, pack_hbm, out_hbm, disv, pk0, pk1, pk2, nb0, nb1, nb2,
          is0, is1, is2, os0, os1, os2):
        c = lax.axis_index("c")
        s = lax.axis_index("s")
        tid = c * NS + s
        q0 = tid * nblk
        e0 = tid * ept
        pks = (pk0, pk1, pk2)
        nbufs = (nb0, nb1, nb2)
        isems = (is0, is1, is2)
        osems = (os0, os1, os2)
        pltpu.sync_copy(dis_hbm, disv)

        z16 = jnp.zeros((L,), jnp.int32)
        one16 = jnp.full((L,), 1, jnp.int32)
        two16 = jnp.full((L,), 2, jnp.int32)
        iota16 = lax.iota(jnp.int32, L)

        def issue_idx(q, b):
            pltpu.async_copy(pack_hbm.at[q0 + q], pks[b], isems[b])

        def wait_idx(q, b):
            pltpu.make_async_copy(pack_hbm.at[q0 + q], pks[b], isems[b]).wait()

        def issue_out(q, b):
            pltpu.async_copy(nbufs[b], out_hbm.at[pl.ds(e0 + q * BLK, BLK)],
                             osems[b])

        def wait_out(q, b):
            pltpu.make_async_copy(nbufs[b],
                                  out_hbm.at[pl.ds(e0 + q * BLK, BLK)],
                                  osems[b]).wait()

        def do_block(q, b, first, pf_idx):
            wait_idx(q, b)
            if pf_idx:
                issue_idx(q + 2, (b + 2) % RING)
            if not first:
                wait_out(q - RING, b)
            for g in range(BLK // L):
                gi = g * L + iota16
                r16 = plsc.load_gather(pks[b], [z16, gi])
                c16 = plsc.load_gather(pks[b], [one16, gi])
                w16 = plsc.bitcast(plsc.load_gather(pks[b], [two16, gi]),
                                   jnp.float32)
                a = plsc.load_gather(disv, [r16])
                bb = plsc.load_gather(disv, [c16])
                nbufs[b][pl.ds(g * L, L)] = a * w16 * bb
            issue_out(q, b)

        issue_idx(0, 0)
        issue_idx(1, 1)
        do_block(0, 0, True, True)
        do_block(1, 1, True, True)
        do_block(2, 2, True, True)

        @pl.loop(1, nblk // RING - 1)
        def _(grp):
            q = grp * RING
            do_block(q, 0, False, True)
            do_block(q + 1, 1, False, True)
            do_block(q + 2, 2, False, True)

        do_block(nblk - 3, 0, False, True)
        do_block(nblk - 2, 1, False, False)
        do_block(nblk - 1, 2, False, False)

        wait_out(nblk - 3, 0)
        wait_out(nblk - 2, 1)
        wait_out(nblk - 1, 2)

    return k(dis, pack)


def _prop_kernel_sync(h, dis, pack, n, e_pad):
    d = h.shape[1]
    ept = e_pad // NTILES
    nblk = ept // BLK

    @functools.partial(
        pl.kernel,
        out_type=jax.ShapeDtypeStruct((NC, n, d), jnp.float32),
        mesh=_MESH,
        compiler_params=_SC_PARAMS,
        scratch_types=[
            pltpu.VMEM((n,), jnp.float32),
            pltpu.VMEM((BLK, d), jnp.float32),
            pltpu.VMEM((3, BLK), jnp.int32),
            pltpu.VMEM((BLK,), jnp.float32),
            pltpu.VMEM_SHARED((n, d), jnp.float32),
        ],
    )
    def k(h_hbm, dis_hbm, pack_hbm, out_hbm, disv, gbuf, pk, nbuf, acc):
        c = lax.axis_index("c")
        s = lax.axis_index("s")
        tid = c * NS + s
        q0 = tid * nblk
        pltpu.sync_copy(dis_hbm, disv)
        _zero_fill(gbuf, BLK, d)
        _zero_acc(gbuf, acc, n, s)
        plsc.subcore_barrier()
        z16 = jnp.zeros((L,), jnp.int32)
        one16 = jnp.full((L,), 1, jnp.int32)
        two16 = jnp.full((L,), 2, jnp.int32)
        iota16 = lax.iota(jnp.int32, L)

        @pl.loop(0, nblk)
        def _(q):
            pltpu.sync_copy(pack_hbm.at[q0 + q], pk)
            pltpu.sync_copy(h_hbm.at[pk.at[0]], gbuf)
            for g in range(BLK // L):
                gi = g * L + iota16
                r16 = plsc.load_gather(pk, [z16, gi])
                c16 = plsc.load_gather(pk, [one16, gi])
                w16 = plsc.bitcast(plsc.load_gather(pk, [two16, gi]), jnp.float32)
                a = plsc.load_gather(disv, [r16])
                bb = plsc.load_gather(disv, [c16])
                nbuf[pl.ds(g * L, L)] = a * w16 * bb

            @pl.loop(0, BLK)
            def _(j):
                nsplat = plsc.load_gather(nbuf, [jnp.full((L,), j, jnp.int32)])
                for k8 in range(d // L):
                    sl = pl.ds(k8 * L, L)
                    gbuf[j, sl] = gbuf[j, sl] * nsplat

            pltpu.sync_copy(gbuf, acc.at[pk.at[1]], add=True)

        plsc.subcore_barrier()
        _dump_acc(acc, out_hbm, c, n, s)

    return k(h, dis, pack)


def _prop_kernel(h, pack, n, e_pad):
    d = h.shape[1]
    ept = e_pad // NTILES
    nblk = ept // BLK
    assert nblk % RING == 0 and nblk // RING >= 2

    @functools.partial(
        pl.kernel,
        out_type=jax.ShapeDtypeStruct((NC, n, d), jnp.float32),
        mesh=_MESH,
        compiler_params=_SC_PARAMS,
        scratch_types=[
            pltpu.VMEM((BLK, d), jnp.float32),
            pltpu.VMEM((BLK, d), jnp.float32),
            pltpu.VMEM((BLK, d), jnp.float32),
            pltpu.VMEM((3, BLK), jnp.int32),
            pltpu.VMEM((3, BLK), jnp.int32),
            pltpu.VMEM((3, BLK), jnp.int32),
            pltpu.SemaphoreType.DMA,
            pltpu.SemaphoreType.DMA,
            pltpu.SemaphoreType.DMA,
            pltpu.SemaphoreType.DMA,
            pltpu.SemaphoreType.DMA,
            pltpu.SemaphoreType.DMA,
            pltpu.SemaphoreType.DMA,
            pltpu.SemaphoreType.DMA,
            pltpu.SemaphoreType.DMA,
            pltpu.VMEM_SHARED((n, d), jnp.float32),
        ],
    )
    def k(h_hbm, pack_hbm, out_hbm, gb0, gb1, gb2, pk0, pk1, pk2,
          is0, is1, is2, gs0, gs1, gs2, ss0, ss1, ss2, acc):
        c = lax.axis_index("c")
        s = lax.axis_index("s")
        tid = c * NS + s
        q0 = tid * nblk
        gbufs = (gb0, gb1, gb2)
        pks = (pk0, pk1, pk2)
        isems = (is0, is1, is2)
        gsems = (gs0, gs1, gs2)
        ssems = (ss0, ss1, ss2)

        _zero_fill(gbufs[0], BLK, d)
        _zero_acc(gbufs[0], acc, n, s)
        plsc.subcore_barrier()

        def issue_idx(q, b):
            pltpu.async_copy(pack_hbm.at[q0 + q], pks[b], isems[b])

        def wait_idx(q, b):
            pltpu.make_async_copy(pack_hbm.at[q0 + q], pks[b], isems[b]).wait()

        def issue_gather(b):
            pltpu.async_copy(h_hbm.at[pks[b].at[0]], gbufs[b], gsems[b])

        def wait_gather(b):
            pltpu.make_async_copy(h_hbm.at[pks[b].at[0]], gbufs[b],
                                  gsems[b]).wait()

        def issue_scatter(b):
            pltpu.async_copy(gbufs[b], acc.at[pks[b].at[1]], ssems[b], add=True)

        def wait_scatter(b):
            pltpu.make_async_copy(gbufs[b], acc.at[pks[b].at[1]],
                                  ssems[b]).wait()

        two16 = jnp.full((L,), 2, jnp.int32)

        def scale(b):
            @pl.loop(0, BLK)
            def _(j):
                ni = plsc.load_gather(pks[b], [two16, jnp.full((L,), j, jnp.int32)])
                nsplat = plsc.bitcast(ni, jnp.float32)
                for k8 in range(d // L):
                    sl = pl.ds(k8 * L, L)
                    gbufs[b][j, sl] = gbufs[b][j, sl] * nsplat

        def do_block(q, b, bn1, bn2, first, last, pf_idx):
            if not last:
                wait_idx(q + 1, bn1)
                issue_gather(bn1)
            wait_gather(b)
            scale(b)
            issue_scatter(b)
            if not first:
                wait_scatter(bn2)
            if pf_idx:
                issue_idx(q + 2, bn2)

        issue_idx(0, 0)
        issue_idx(1, 1)
        wait_idx(0, 0)
        issue_gather(0)
        do_block(0, 0, 1, 2, True, False, True)
        do_block(1, 1, 2, 0, False, False, True)
        do_block(2, 2, 0, 1, False, False, True)

        @pl.loop(1, nblk // RING - 1)
        def _(grp):
            q = grp * RING
            do_block(q, 0, 1, 2, False, False, True)
            do_block(q + 1, 1, 2, 0, False, False, True)
            do_block(q + 2, 2, 0, 1, False, False, True)

        do_block(nblk - 3, 0, 1, 2, False, False, True)
        do_block(nblk - 2, 1, 2, 0, False, False, False)
        do_block(nblk - 1, 2, 0, 1, False, True, False)

        wait_scatter(2)
        plsc.subcore_barrier()
        _dump_acc(acc, out_hbm, c, n, s)

    return k(h, pack)


def _matmul(x, w):
    n, d = x.shape
    blk = 1000

    def body(x_ref, w_ref, o_ref):
        o_ref[...] = jnp.dot(x_ref[...], w_ref[...],
                             preferred_element_type=jnp.float32)

    return pl.pallas_call(
        body,
        out_shape=jax.ShapeDtypeStruct((n, d), jnp.float32),
        grid=(n // blk,),
        in_specs=[
            pl.BlockSpec((blk, d), lambda i: (i, 0)),
            pl.BlockSpec((d, d), lambda i: (0, 0)),
        ],
        out_specs=pl.BlockSpec((blk, d), lambda i: (i, 0)),
    )(x, w)


def _ln_kernel(p, b):
    _, n, d = p.shape
    blk = 1000

    def body(p_ref, b_ref, o_ref):
        t = p_ref[0] + p_ref[1] + b_ref[...]
        mu = jnp.mean(t, axis=-1, keepdims=True)
        var = jnp.mean((t - mu) ** 2, axis=-1, keepdims=True)
        y = (t - mu) * lax.rsqrt(var + LN_EPS)
        o_ref[...] = jnp.maximum(y, 0.0)

    return pl.pallas_call(
        body,
        out_shape=jax.ShapeDtypeStruct((n, d), jnp.float32),
        grid=(n // blk,),
        in_specs=[
            pl.BlockSpec((2, blk, d), lambda i: (0, i, 0)),
            pl.BlockSpec((1, d), lambda i: (0, 0)),
        ],
        out_specs=pl.BlockSpec((blk, d), lambda i: (i, 0)),
    )(p, b)


def kernel(x, edge_index, edge_weight, W1, b1, W2, b2):
    n, d = x.shape
    e = edge_weight.shape[0]
    row = edge_index[0].astype(jnp.int32)
    col = edge_index[1].astype(jnp.int32)
    loop_idx = jnp.arange(n, dtype=jnp.int32)
    e_full = e + n
    chunk = NTILES * BLK * RING
    e_pad = ((e_full + chunk - 1) // chunk) * chunk
    pad = e_pad - e_full
    rowf = jnp.concatenate([row, loop_idx, jnp.zeros((pad,), jnp.int32)])
    colf = jnp.concatenate([col, loop_idx, jnp.zeros((pad,), jnp.int32)])
    wf = jnp.concatenate([edge_weight.astype(jnp.float32),
                          jnp.ones((n,), jnp.float32),
                          jnp.zeros((pad,), jnp.float32)])

    nb_tot = e_pad // BLK
    rowb = rowf.reshape(nb_tot, BLK)
    colb = colf.reshape(nb_tot, BLK)
    pack1 = jnp.stack([rowb, colb,
                       lax.bitcast_convert_type(wf, jnp.int32)
                          .reshape(nb_tot, BLK)], axis=1)

    degp = _deg_pipe(pack1, n, e_pad)
    dis = _dis_kernel(degp)
    normf = _norm_pipe(dis, pack1, e_pad)
    pack2 = jnp.stack([rowb, colb,
                       lax.bitcast_convert_type(normf, jnp.int32)
                          .reshape(nb_tot, BLK)], axis=1)

    h = _matmul(x, W1)
    p = _prop_kernel(h, pack2, n, e_pad)
    h = _ln_kernel(p, b1.reshape(1, d))
    h = _matmul(h, W2)
    p = _prop_kernel(h, pack2, n, e_pad)
    h = _ln_kernel(p, b2.reshape(1, d))
    return h
